# Initial kernel scaffold; baseline (speedup 1.0000x reference)
import jax
import jax.numpy as jnp
from jax.experimental import pallas as pl


def kernel(pos, feat, cluster_ids, W1a, b1a, W2a, b2a, W1b, b1b, W2b, b2b):
    raise NotImplementedError("write your pallas kernel here")


# speedup vs baseline: 4.7272x; 4.7272x over previous
import functools

import jax
import jax.numpy as jnp
from jax import lax
from jax.experimental import pallas as pl
from jax.experimental.pallas import tpu as pltpu
from jax.experimental.pallas import tpu_sc as plsc

B, N, D = 4, 65536, 64
NC = 8192
NPTS = B * N
NSEG = B * NC

SC_CORES = 2
SC_SUBCORES = 16

SCAT_PTS = NPTS // SC_SUBCORES
GATH_PTS = NPTS // (SC_CORES * SC_SUBCORES)
ZERO_ROWS = NSEG // SC_SUBCORES
CHUNK = 128
SCAT_CHUNKS = SCAT_PTS // CHUNK
GATH_CHUNKS = GATH_PTS // CHUNK
RW = 8


def _sc_cog(rec, seg2d, zeros_tab, interpret=False):
    mesh = plsc.VectorSubcoreMesh(core_axis_name="c", subcore_axis_name="s")

    @functools.partial(
        pl.kernel,
        out_type=jax.ShapeDtypeStruct((NPTS, RW), jnp.float32),
        mesh=mesh,
        scratch_types=[
            pltpu.VMEM((SCAT_CHUNKS, CHUNK), jnp.int32),
            pltpu.VMEM((GATH_PTS, RW), jnp.float32),
            pltpu.VMEM_SHARED((NSEG, RW), jnp.float32),
        ],
        compiler_params=pltpu.CompilerParams(use_tc_tiling_on_sc=False),
        interpret=interpret,
    )
    def k(rec_hbm, seg_hbm, z_hbm, cogpt_hbm, idx_v, rec_v, table):
        c = lax.axis_index("c")
        s = lax.axis_index("s")
        pltpu.sync_copy(z_hbm.at[pl.ds(s * ZERO_ROWS, ZERO_ROWS), :],
                        table.at[pl.ds(s * ZERO_ROWS, ZERO_ROWS), :])
        pltpu.sync_copy(seg_hbm.at[pl.ds(s * SCAT_CHUNKS, SCAT_CHUNKS), :],
                        idx_v)
        plsc.subcore_barrier()

        for p in range(SCAT_PTS // GATH_PTS):
            pltpu.sync_copy(
                rec_hbm.at[pl.ds(s * SCAT_PTS + p * GATH_PTS, GATH_PTS), :],
                rec_v)

            def scat(j, carry, _p=p):
                pltpu.sync_copy(rec_v.at[pl.ds(j * CHUNK, CHUNK), :],
                                table.at[idx_v.at[_p * GATH_CHUNKS + j]],
                                add=True)
                return carry
            lax.fori_loop(0, GATH_CHUNKS, scat, 0)
        plsc.subcore_barrier()

        gbase = c * (NPTS // SC_CORES) + s * GATH_PTS
        grow = c * (NPTS // SC_CORES // CHUNK) + s * GATH_CHUNKS
        pltpu.sync_copy(seg_hbm.at[pl.ds(grow, GATH_CHUNKS), :],
                        idx_v.at[pl.ds(0, GATH_CHUNKS), :])

        def gath(j, carry):
            pltpu.sync_copy(table.at[idx_v.at[j]],
                            rec_v.at[pl.ds(j * CHUNK, CHUNK), :])
            return carry
        lax.fori_loop(0, GATH_CHUNKS, gath, 0)
        pltpu.sync_copy(rec_v.at[pl.ds(0, GATH_PTS), :],
                        cogpt_hbm.at[pl.ds(gbase, GATH_PTS), :])

    return k(rec, seg2d, zeros_tab)


def _tc_mlp(rec, cogpt, feat2d, W1a, b1a, W2a, b2a, W1bg, b1b, W2b, b2b,
            interpret=False):
    P = 2048
    grid = (NPTS // P,)

    def body(rec_ref, cog_ref, feat_ref, w1a, b1a_r, w2a, b2a_r,
             w1b, b1b_r, w2b, b2b_r, o1, o2):
        rec_b = rec_ref[...]
        cp = cog_ref[...]
        inv = 1.0 / jnp.maximum(cp[:, 3:4], 1.0)
        lp0 = rec_b[:, 0:1] - cp[:, 0:1] * inv
        lp1 = rec_b[:, 1:2] - cp[:, 1:2] * inv
        lp2 = rec_b[:, 2:3] - cp[:, 2:3] * inv
        nrm = jnp.sqrt(lp0 * lp0 + lp1 * lp1 + lp2 * lp2)
        w1a_v = w1a[...]
        h1 = jnp.maximum(
            lp0 * w1a_v[0:1] + lp1 * w1a_v[1:2] + lp2 * w1a_v[2:3]
            + nrm * w1a_v[3:4] + b1a_r[...], 0.0)
        f = feat_ref[...]
        o1[...] = f + jnp.dot(h1, w2a[...],
                              preferred_element_type=jnp.float32) + b2a_r[...]
        w1b_v = w1b[...]
        h2 = jnp.maximum(
            lp0 * w1b_v[0:1] + lp1 * w1b_v[1:2] + lp2 * w1b_v[2:3]
            + b1b_r[...], 0.0)
        o2[...] = f + jnp.dot(h2, w2b[...],
                              preferred_element_type=jnp.float32) + b2b_r[...]

    def wspec(shape):
        return pl.BlockSpec(shape, lambda i: (0,) * len(shape))

    return pl.pallas_call(
        body,
        grid=grid,
        in_specs=[
            pl.BlockSpec((P, RW), lambda i: (i, 0)),
            pl.BlockSpec((P, RW), lambda i: (i, 0)),
            pl.BlockSpec((P, D), lambda i: (i, 0)),
            wspec((4, D)), wspec((1, D)), wspec((D, D)), wspec((1, D)),
            wspec((3, D)), wspec((1, D)), wspec((D, D)), wspec((1, D)),
        ],
        out_specs=[
            pl.BlockSpec((P, D), lambda i: (i, 0)),
            pl.BlockSpec((P, D), lambda i: (i, 0)),
        ],
        out_shape=[
            jax.ShapeDtypeStruct((NPTS, D), jnp.float32),
            jax.ShapeDtypeStruct((NPTS, D), jnp.float32),
        ],
        interpret=interpret,
    )(rec, cogpt, feat2d, W1a, b1a, W2a, b2a, W1bg, b1b, W2b, b2b)


def kernel(pos, feat, cluster_ids, W1a, b1a, W2a, b2a, W1b, b1b, W2b, b2b):
    posf = pos.reshape(NPTS, 3)
    rec = jnp.concatenate(
        [posf, jnp.ones((NPTS, 1), jnp.float32),
         jnp.zeros((NPTS, RW - 4), jnp.float32)], axis=1)
    seg = (cluster_ids
           + (jnp.arange(B, dtype=jnp.int32) * NC)[:, None]).reshape(NPTS)
    zeros_tab = jnp.zeros((NSEG, RW), jnp.float32)
    seg2d = seg.reshape(NPTS // CHUNK, CHUNK)
    cogpt = _sc_cog(rec, seg2d, zeros_tab)
    feat2d = feat.reshape(NPTS, D)
    o1, o2 = _tc_mlp(rec, cogpt, feat2d,
                     W1a, b1a.reshape(1, D), W2a, b2a.reshape(1, D),
                     W1b[3:6], b1b.reshape(1, D), W2b, b2b.reshape(1, D))
    return o1.reshape(B, N, D), o2.reshape(B, N, D)


# speedup vs baseline: 5.7725x; 1.2211x over previous
import functools

import jax
import jax.numpy as jnp
from jax import lax
from jax.experimental import pallas as pl
from jax.experimental.pallas import tpu as pltpu
from jax.experimental.pallas import tpu_sc as plsc

B, N, D = 4, 65536, 64
NC = 8192
NPTS = B * N
NSEG = B * NC

SC_CORES = 2
SC_SUBCORES = 16

SCAT_PTS = NPTS // SC_SUBCORES
GATH_PTS = NPTS // (SC_CORES * SC_SUBCORES)
ZERO_ROWS = NSEG // SC_SUBCORES
CHUNK = 128
SCAT_CHUNKS = SCAT_PTS // CHUNK
GATH_CHUNKS = GATH_PTS // CHUNK
RW = 8


def _sc_cog(rec, seg2d, zeros_tab, interpret=False):
    mesh = plsc.VectorSubcoreMesh(core_axis_name="c", subcore_axis_name="s")

    @functools.partial(
        pl.kernel,
        out_type=jax.ShapeDtypeStruct((NPTS, RW), jnp.float32),
        mesh=mesh,
        scratch_types=[
            pltpu.VMEM((SCAT_CHUNKS, CHUNK), jnp.int32),
            pltpu.VMEM((GATH_PTS, RW), jnp.float32),
            pltpu.VMEM_SHARED((NSEG, RW), jnp.float32),
        ],
        compiler_params=pltpu.CompilerParams(use_tc_tiling_on_sc=False),
        interpret=interpret,
    )
    def k(rec_hbm, seg_hbm, z_hbm, cogpt_hbm, idx_v, rec_v, table):
        c = lax.axis_index("c")
        s = lax.axis_index("s")
        pltpu.sync_copy(z_hbm.at[pl.ds(s * ZERO_ROWS, ZERO_ROWS), :],
                        table.at[pl.ds(s * ZERO_ROWS, ZERO_ROWS), :])
        pltpu.sync_copy(seg_hbm.at[pl.ds(s * SCAT_CHUNKS, SCAT_CHUNKS), :],
                        idx_v)
        plsc.subcore_barrier()

        for p in range(SCAT_PTS // GATH_PTS):
            pltpu.sync_copy(
                rec_hbm.at[pl.ds(s * SCAT_PTS + p * GATH_PTS, GATH_PTS), :],
                rec_v)

            def scat(j, carry, _p=p):
                pltpu.sync_copy(rec_v.at[pl.ds(j * CHUNK, CHUNK), :],
                                table.at[idx_v.at[_p * GATH_CHUNKS + j]],
                                add=True)
                return carry
            lax.fori_loop(0, GATH_CHUNKS, scat, 0)
        plsc.subcore_barrier()

        gbase = c * (NPTS // SC_CORES) + s * GATH_PTS
        grow = c * (NPTS // SC_CORES // CHUNK) + s * GATH_CHUNKS
        pltpu.sync_copy(seg_hbm.at[pl.ds(grow, GATH_CHUNKS), :],
                        idx_v.at[pl.ds(0, GATH_CHUNKS), :])

        def gath(j, carry):
            pltpu.sync_copy(table.at[idx_v.at[j]],
                            rec_v.at[pl.ds(j * CHUNK, CHUNK), :])
            return carry
        lax.fori_loop(0, GATH_CHUNKS, gath, 0)
        pltpu.sync_copy(rec_v.at[pl.ds(0, GATH_PTS), :],
                        cogpt_hbm.at[pl.ds(gbase, GATH_PTS), :])

    return k(rec, seg2d, zeros_tab)


def _tc_mlp(rec, cogpt, feat2d, W1a8, b1a, W2a, b2a, W1b8, b1b, W2b, b2b,
            interpret=False):
    P = 4096
    grid = (NPTS // P,)

    def body(rec_ref, cog_ref, feat_ref, w1a, b1a_r, w2a, b2a_r,
             w1b, b1b_r, w2b, b2b_r, o1, o2):
        rec_b = rec_ref[...]
        cp = cog_ref[...]
        inv = 1.0 / jnp.maximum(cp[:, 3:4], 1.0)
        lp8 = rec_b - cp * inv
        nrm = jnp.sqrt(jnp.sum(lp8 * lp8, axis=1, keepdims=True))
        col = lax.broadcasted_iota(jnp.int32, (1, RW), 1)
        x1 = lp8 + nrm * (col == 3).astype(jnp.float32)
        h1 = jnp.maximum(
            jnp.dot(x1, w1a[...], preferred_element_type=jnp.float32)
            + b1a_r[...], 0.0)
        f = feat_ref[...]
        o1[...] = f + jnp.dot(h1, w2a[...],
                              preferred_element_type=jnp.float32) + b2a_r[...]
        h2 = jnp.maximum(
            jnp.dot(lp8, w1b[...], preferred_element_type=jnp.float32)
            + b1b_r[...], 0.0)
        o2[...] = f + jnp.dot(h2, w2b[...],
                              preferred_element_type=jnp.float32) + b2b_r[...]

    def wspec(shape):
        return pl.BlockSpec(shape, lambda i: (0,) * len(shape))

    return pl.pallas_call(
        body,
        grid=grid,
        in_specs=[
            pl.BlockSpec((P, RW), lambda i: (i, 0)),
            pl.BlockSpec((P, RW), lambda i: (i, 0)),
            pl.BlockSpec((P, D), lambda i: (i, 0)),
            wspec((RW, D)), wspec((1, D)), wspec((D, D)), wspec((1, D)),
            wspec((RW, D)), wspec((1, D)), wspec((D, D)), wspec((1, D)),
        ],
        out_specs=[
            pl.BlockSpec((P, D), lambda i: (i, 0)),
            pl.BlockSpec((P, D), lambda i: (i, 0)),
        ],
        out_shape=[
            jax.ShapeDtypeStruct((NPTS, D), jnp.float32),
            jax.ShapeDtypeStruct((NPTS, D), jnp.float32),
        ],
        interpret=interpret,
    )(rec, cogpt, feat2d, W1a8, b1a, W2a, b2a, W1b8, b1b, W2b, b2b)


def kernel(pos, feat, cluster_ids, W1a, b1a, W2a, b2a, W1b, b1b, W2b, b2b):
    posf = pos.reshape(NPTS, 3)
    rec = jnp.concatenate(
        [posf, jnp.ones((NPTS, 1), jnp.float32),
         jnp.zeros((NPTS, RW - 4), jnp.float32)], axis=1)
    seg = (cluster_ids
           + (jnp.arange(B, dtype=jnp.int32) * NC)[:, None]).reshape(NPTS)
    zeros_tab = jnp.zeros((NSEG, RW), jnp.float32)
    seg2d = seg.reshape(NPTS // CHUNK, CHUNK)
    cogpt = _sc_cog(rec, seg2d, zeros_tab)
    feat2d = feat.reshape(NPTS, D)
    W1a8 = jnp.concatenate([W1a, jnp.zeros((RW - 4, D), jnp.float32)], axis=0)
    W1b8 = jnp.concatenate([W1b[3:6], jnp.zeros((RW - 3, D), jnp.float32)],
                           axis=0)
    o1, o2 = _tc_mlp(rec, cogpt, feat2d,
                     W1a8, b1a.reshape(1, D), W2a, b2a.reshape(1, D),
                     W1b8, b1b.reshape(1, D), W2b, b2b.reshape(1, D))
    return o1.reshape(B, N, D), o2.reshape(B, N, D)
